# baseline (speedup 1.0000x reference)
import functools
import jax
import jax.numpy as jnp
from jax import lax
from jax.experimental import pallas as pl
from jax.experimental.pallas import tpu as pltpu
from jax.experimental.pallas import tpu_sc as plsc

NC, NS = 2, 16
NW = NC * NS
D = 50
DP = 64
B = 4096 * 200
CHUNK = 128
GROUP = 4
GROW = CHUNK * GROUP
ROWS_PER_W = B // NW
NGROUPS = ROWS_PER_W // GROW
NBLK = GROW * D // 400

_mesh = plsc.VectorSubcoreMesh(core_axis_name="c", subcore_axis_name="s")


@functools.partial(
    pl.kernel,
    out_type=jax.ShapeDtypeStruct((B * D,), jnp.float32),
    mesh=_mesh,
    scratch_types=[
        pltpu.VMEM((GROUP, CHUNK), jnp.int32),
        pltpu.VMEM((GROW, DP), jnp.float32),
        pltpu.VMEM((GROW * D,), jnp.float32),
        pltpu.SemaphoreType.DMA,
    ],
    compiler_params=pltpu.CompilerParams(use_tc_tiling_on_sc=False, needs_layout_passes=False),
)
def _emb_lookup(ids_hbm, table_hbm, out_hbm, idx_v, pad_v, cmp_v, gsem):
    wid = lax.axis_index("s") * NC + lax.axis_index("c")
    row_base = wid * ROWS_PER_W

    lane = lax.broadcasted_iota(jnp.int32, (16,), 0)
    pat = []
    for k in range(25):
        p = k * 16 + lane
        pat.append((p // D, p % D))

    def body(g, carry):
        pltpu.sync_copy(ids_hbm.at[wid, g], idx_v)
        copies = []
        for j in range(GROUP):
            copies.append(
                pltpu.async_copy(
                    table_hbm.at[idx_v.at[j]],
                    pad_v.at[pl.ds(j * CHUNK, CHUNK)],
                    gsem,
                )
            )
        for c in copies:
            c.wait()

        def blk(b2, carry2):
            base_row = b2 * 8
            off0 = b2 * 400
            for k in range(25):
                v = plsc.load_gather(pad_v, [pat[k][0] + base_row, pat[k][1]])
                cmp_v[pl.ds(off0 + k * 16, 16)] = v
            return carry2

        lax.fori_loop(0, NBLK, blk, 0)

        pltpu.sync_copy(
            cmp_v,
            out_hbm.at[pl.ds((row_base + g * GROW) * D, GROW * D)],
        )
        return carry

    lax.fori_loop(0, NGROUPS, body, 0)


def kernel(pos_ids, table):
    ids = pos_ids.reshape(NW, NGROUPS, GROUP, CHUNK)
    table_p = jnp.pad(table, ((0, 0), (0, DP - D)))
    out = _emb_lookup(ids, table_p)
    return out.reshape(pos_ids.shape[0], pos_ids.shape[1], D)


# speedup vs baseline: 1.2089x; 1.2089x over previous
import functools
import jax
import jax.numpy as jnp
from jax import lax
from jax.experimental import pallas as pl
from jax.experimental.pallas import tpu as pltpu
from jax.experimental.pallas import tpu_sc as plsc

NC, NS = 2, 16
NW = NC * NS
D = 50
DP = 64
B = 4096 * 200
CHUNK = 128
GROUP = 4
GROW = CHUNK * GROUP
ROWS_PER_W = B // NW
NGROUPS = ROWS_PER_W // GROW
NBLK = GROW * D // 400

_mesh = plsc.VectorSubcoreMesh(core_axis_name="c", subcore_axis_name="s")


@functools.partial(
    pl.kernel,
    out_type=jax.ShapeDtypeStruct((B * D,), jnp.float32),
    mesh=_mesh,
    scratch_types=[
        pltpu.VMEM((2, GROUP, CHUNK), jnp.int32),
        pltpu.VMEM((2, GROW, DP), jnp.float32),
        pltpu.VMEM((2, GROW * D), jnp.float32),
        pltpu.SemaphoreType.DMA,
        pltpu.SemaphoreType.DMA,
        pltpu.SemaphoreType.DMA,
        pltpu.SemaphoreType.DMA,
        pltpu.SemaphoreType.DMA,
        pltpu.SemaphoreType.DMA,
    ],
    compiler_params=pltpu.CompilerParams(
        use_tc_tiling_on_sc=False, needs_layout_passes=False
    ),
)
def _emb_lookup(ids_hbm, table_hbm, out_hbm, idx_v, pad_v, cmp_v,
                isem0, isem1, gsem0, gsem1, osem0, osem1):
    wid = lax.axis_index("s") * NC + lax.axis_index("c")
    row_base = wid * ROWS_PER_W
    isem = (isem0, isem1)
    gsem = (gsem0, gsem1)
    osem = (osem0, osem1)

    lane = lax.broadcasted_iota(jnp.int32, (16,), 0)
    pat = []
    for k in range(25):
        p = k * 16 + lane
        pat.append((p // D, p % D))

    def idx_issue(g, b):
        pltpu.async_copy(ids_hbm.at[wid, g], idx_v.at[b], isem[b])

    def idx_wait(g, b):
        pltpu.make_async_copy(ids_hbm.at[wid, g], idx_v.at[b], isem[b]).wait()

    def gather_fire(b):
        for j in range(GROUP):
            pltpu.async_copy(
                table_hbm.at[idx_v.at[b, j]],
                pad_v.at[b, pl.ds(j * CHUNK, CHUNK)],
                gsem[b],
            )

    def gather_drain(b):
        for j in range(GROUP):
            pltpu.make_async_copy(
                table_hbm.at[idx_v.at[b, j]],
                pad_v.at[b, pl.ds(j * CHUNK, CHUNK)],
                gsem[b],
            ).wait()

    def compact(b):
        @plsc.parallel_loop(0, NBLK, 1, unroll=2)
        def blk(b2):
            base_row = b2 * 8
            off0 = b2 * 400
            for k in range(25):
                v = plsc.load_gather(
                    pad_v.at[b], [pat[k][0] + base_row, pat[k][1]]
                )
                cmp_v[b, pl.ds(off0 + k * 16, 16)] = v

    def write_issue(g, b):
        pltpu.async_copy(
            cmp_v.at[b],
            out_hbm.at[pl.ds((row_base + g * GROW) * D, GROW * D)],
            osem[b],
        )

    def write_wait(b):
        pltpu.make_async_copy(
            cmp_v.at[b],
            out_hbm.at[pl.ds(row_base * D, GROW * D)],
            osem[b],
        ).wait()

    idx_issue(0, 0)
    idx_issue(1, 1)
    idx_wait(0, 0)
    gather_fire(0)

    def body(gg, carry):
        ga = 2 * gg
        gb = ga + 1
        idx_wait(gb, 1)
        gather_fire(1)
        gather_drain(0)

        @pl.when(ga + 2 < NGROUPS)
        def _():
            idx_issue(ga + 2, 0)

        @pl.when(gg >= 1)
        def _():
            write_wait(0)

        compact(0)
        write_issue(ga, 0)

        @pl.when(ga + 2 < NGROUPS)
        def _():
            idx_wait(ga + 2, 0)
            gather_fire(0)

        gather_drain(1)

        @pl.when(gb + 2 < NGROUPS)
        def _():
            idx_issue(gb + 2, 1)

        @pl.when(gg >= 1)
        def _():
            write_wait(1)

        compact(1)
        write_issue(gb, 1)
        return carry

    lax.fori_loop(0, NGROUPS // 2, body, 0)
    write_wait(0)
    write_wait(1)


def kernel(pos_ids, table):
    ids = pos_ids.reshape(NW, NGROUPS, GROUP, CHUNK)
    table_p = jnp.pad(table, ((0, 0), (0, DP - D)))
    out = _emb_lookup(ids, table_p)
    return out.reshape(pos_ids.shape[0], pos_ids.shape[1], D)


# speedup vs baseline: 1.2127x; 1.0031x over previous
import functools
import jax
import jax.numpy as jnp
from jax import lax
from jax.experimental import pallas as pl
from jax.experimental.pallas import tpu as pltpu
from jax.experimental.pallas import tpu_sc as plsc

NC, NS = 2, 16
NW = NC * NS
D = 50
DP = 64
B = 4096 * 200
CHUNK = 128
GROUP = 4
GROW = CHUNK * GROUP
ROWS_PER_W = B // NW
NGROUPS = ROWS_PER_W // GROW
NBLK = GROW * D // 400

_mesh = plsc.VectorSubcoreMesh(core_axis_name="c", subcore_axis_name="s")


@functools.partial(
    pl.kernel,
    out_type=jax.ShapeDtypeStruct((B * D,), jnp.float32),
    mesh=_mesh,
    scratch_types=[
        pltpu.VMEM((2, GROUP, CHUNK), jnp.int32),
        pltpu.VMEM((2, GROW, DP), jnp.float32),
        pltpu.VMEM((2, GROW * D), jnp.float32),
        pltpu.SemaphoreType.DMA,
        pltpu.SemaphoreType.DMA,
        pltpu.SemaphoreType.DMA,
        pltpu.SemaphoreType.DMA,
        pltpu.SemaphoreType.DMA,
        pltpu.SemaphoreType.DMA,
    ],
    compiler_params=pltpu.CompilerParams(
        use_tc_tiling_on_sc=False, needs_layout_passes=False, disable_bounds_checks=True
    ),
)
def _emb_lookup(ids_hbm, table_hbm, out_hbm, idx_v, pad_v, cmp_v,
                isem0, isem1, gsem0, gsem1, osem0, osem1):
    wid = lax.axis_index("s") * NC + lax.axis_index("c")
    row_base = wid * ROWS_PER_W
    isem = (isem0, isem1)
    gsem = (gsem0, gsem1)
    osem = (osem0, osem1)

    lane = lax.broadcasted_iota(jnp.int32, (16,), 0)
    pat = []
    for k in range(25):
        p = k * 16 + lane
        pat.append((p // D, p % D))

    def idx_issue(g, b):
        pltpu.async_copy(ids_hbm.at[wid, g], idx_v.at[b], isem[b])

    def idx_wait(g, b):
        pltpu.make_async_copy(ids_hbm.at[wid, g], idx_v.at[b], isem[b]).wait()

    def gather_fire(b):
        for j in range(GROUP):
            pltpu.async_copy(
                table_hbm.at[idx_v.at[b, j]],
                pad_v.at[b, pl.ds(j * CHUNK, CHUNK)],
                gsem[b],
            )

    def gather_drain(b):
        for j in range(GROUP):
            pltpu.make_async_copy(
                table_hbm.at[idx_v.at[b, j]],
                pad_v.at[b, pl.ds(j * CHUNK, CHUNK)],
                gsem[b],
            ).wait()

    def compact(b):
        @plsc.parallel_loop(0, NBLK, 1, unroll=4)
        def blk(b2):
            base_row = b2 * 8
            off0 = b2 * 400
            for k in range(25):
                v = plsc.load_gather(
                    pad_v.at[b], [pat[k][0] + base_row, pat[k][1]]
                )
                cmp_v[b, pl.ds(off0 + k * 16, 16)] = v

    def write_issue(g, b):
        pltpu.async_copy(
            cmp_v.at[b],
            out_hbm.at[pl.ds((row_base + g * GROW) * D, GROW * D)],
            osem[b],
        )

    def write_wait(b):
        pltpu.make_async_copy(
            cmp_v.at[b],
            out_hbm.at[pl.ds(row_base * D, GROW * D)],
            osem[b],
        ).wait()

    idx_issue(0, 0)
    idx_issue(1, 1)
    idx_wait(0, 0)
    gather_fire(0)

    def body(gg, carry):
        ga = 2 * gg
        gb = ga + 1
        idx_wait(gb, 1)
        gather_fire(1)
        gather_drain(0)

        @pl.when(ga + 2 < NGROUPS)
        def _():
            idx_issue(ga + 2, 0)

        @pl.when(gg >= 1)
        def _():
            write_wait(0)

        compact(0)
        write_issue(ga, 0)

        @pl.when(ga + 2 < NGROUPS)
        def _():
            idx_wait(ga + 2, 0)
            gather_fire(0)

        gather_drain(1)

        @pl.when(gb + 2 < NGROUPS)
        def _():
            idx_issue(gb + 2, 1)

        @pl.when(gg >= 1)
        def _():
            write_wait(1)

        compact(1)
        write_issue(gb, 1)
        return carry

    lax.fori_loop(0, NGROUPS // 2, body, 0)
    write_wait(0)
    write_wait(1)


def kernel(pos_ids, table):
    ids = pos_ids.reshape(NW, NGROUPS, GROUP, CHUNK)
    table_p = jnp.pad(table, ((0, 0), (0, DP - D)))
    out = _emb_lookup(ids, table_p)
    return out.reshape(pos_ids.shape[0], pos_ids.shape[1], D)
